# baseline (speedup 1.0000x reference)
import functools

import jax
import jax.numpy as jnp
from jax import lax
from jax.experimental import pallas as pl
from jax.experimental.pallas import tpu as pltpu
from jax.experimental.pallas import tpu_sc as plsc

NUM_CORES = 2
NUM_SUBCORES = 16
LOOKAHEAD = 4
NBUF = 2 * LOOKAHEAD


def kernel(input_ids, table):
    batch, seq = input_ids.shape
    vocab, hidden = table.shape
    num_idx = batch * seq
    flat_idx = input_ids.reshape(num_idx)

    chunk = 40
    num_workers = NUM_CORES * NUM_SUBCORES
    b_per_worker = batch // num_workers
    per_worker = num_idx // num_workers
    n_chunks = per_worker // chunk
    steady = n_chunks - 2 * LOOKAHEAD
    n_outer = steady // NBUF
    assert steady == n_outer * NBUF

    mesh = plsc.VectorSubcoreMesh(core_axis_name="c", subcore_axis_name="s")

    @functools.partial(
        pl.kernel,
        mesh=mesh,
        out_type=jax.ShapeDtypeStruct((batch, seq, hidden), table.dtype),
        scratch_types=[
            pltpu.VMEM((per_worker,), jnp.int32),
        ]
        + [pltpu.VMEM((chunk, hidden), jnp.float32)] * NBUF
        + [pltpu.SemaphoreType.DMA] * (2 * NBUF)
        + [pltpu.SemaphoreType.DMA],
        compiler_params=pltpu.CompilerParams(use_tc_tiling_on_sc=False),
    )
    def gather_kernel(table_hbm, idx_hbm, out_hbm, idx_all, *rest):
        bufs = rest[:NBUF]
        g_sem = rest[NBUF:2 * NBUF]
        s_sem = rest[2 * NBUF:3 * NBUF]
        idx_sem = rest[3 * NBUF]

        wid = lax.axis_index("s") * NUM_CORES + lax.axis_index("c")
        base_w = wid * per_worker
        base_b = wid * b_per_worker

        pltpu.async_copy(
            idx_hbm.at[pl.ds(base_w, per_worker)], idx_all, idx_sem).wait()

        def idx_slice(t):
            return idx_all.at[pl.ds(t * chunk, chunk)]

        chunks_per_row = seq // chunk

        def out_slice(t):
            return out_hbm.at[
                base_b + t // chunks_per_row,
                pl.ds((t % chunks_per_row) * chunk, chunk),
            ]

        def gather_start(t, b):
            pltpu.make_async_copy(
                table_hbm.at[idx_slice(t)], bufs[b], g_sem[b]).start()

        def gather_wait(t, b):
            pltpu.make_async_copy(
                table_hbm.at[idx_slice(t)], bufs[b], g_sem[b]).wait()

        def store_start(t, b):
            pltpu.make_async_copy(bufs[b], out_slice(t), s_sem[b]).start()

        def store_wait(t, b):
            pltpu.make_async_copy(bufs[b], out_slice(t), s_sem[b]).wait()

        for t in range(LOOKAHEAD):
            gather_start(t, t)
        for t in range(LOOKAHEAD):
            gather_wait(t, t)
            store_start(t, t)
            gather_start(t + LOOKAHEAD, t + LOOKAHEAD)

        @pl.loop(0, n_outer)
        def _(k):
            for j in range(NBUF):
                t = LOOKAHEAD + k * NBUF + j
                b = (LOOKAHEAD + j) % NBUF
                b2 = (b + LOOKAHEAD) % NBUF
                gather_wait(t, b)
                store_start(t, b)
                store_wait(t - LOOKAHEAD, b2)
                gather_start(t + LOOKAHEAD, b2)

        for i in range(LOOKAHEAD):
            t = n_chunks - LOOKAHEAD + i
            b = t % NBUF
            gather_wait(t, b)
            store_start(t, b)

        for i in range(NBUF):
            t = n_chunks - NBUF + i
            store_wait(t, t % NBUF)

    return gather_kernel(table, flat_idx)


# speedup vs baseline: 1.0174x; 1.0174x over previous
import functools

import jax
import jax.numpy as jnp
from jax import lax
from jax.experimental import pallas as pl
from jax.experimental.pallas import tpu as pltpu
from jax.experimental.pallas import tpu_sc as plsc

NUM_CORES = 2
NUM_SUBCORES = 16
CHUNK = 128
LOOKAHEAD = 4
NBUF = 2 * LOOKAHEAD


def kernel(input_ids, table):
    batch, seq = input_ids.shape
    vocab, hidden = table.shape
    num_idx = batch * seq
    flat_idx = input_ids.reshape(num_idx)

    num_workers = NUM_CORES * NUM_SUBCORES
    per_worker = num_idx // num_workers
    n_chunks = per_worker // CHUNK
    steady = n_chunks - 2 * LOOKAHEAD
    n_outer = steady // NBUF
    assert steady == n_outer * NBUF

    mesh = plsc.VectorSubcoreMesh(core_axis_name="c", subcore_axis_name="s")

    @functools.partial(
        pl.kernel,
        mesh=mesh,
        out_type=jax.ShapeDtypeStruct((num_idx, hidden), table.dtype),
        scratch_types=[
            pltpu.VMEM((per_worker,), jnp.int32),
        ]
        + [pltpu.VMEM((CHUNK, hidden), jnp.float32)] * NBUF
        + [pltpu.SemaphoreType.DMA] * (2 * NBUF)
        + [pltpu.SemaphoreType.DMA],
        compiler_params=pltpu.CompilerParams(use_tc_tiling_on_sc=False),
    )
    def gather_kernel(table_hbm, idx_hbm, out_hbm, idx_all, *rest):
        bufs = rest[:NBUF]
        g_sem = rest[NBUF:2 * NBUF]
        s_sem = rest[2 * NBUF:3 * NBUF]
        idx_sem = rest[3 * NBUF]

        wid = lax.axis_index("s") * NUM_CORES + lax.axis_index("c")
        base_w = wid * per_worker

        pltpu.async_copy(
            idx_hbm.at[pl.ds(base_w, per_worker)], idx_all, idx_sem).wait()

        def idx_slice(t):
            return idx_all.at[pl.ds(t * CHUNK, CHUNK)]

        def out_slice(t):
            return out_hbm.at[pl.ds(base_w + t * CHUNK, CHUNK)]

        def gather_start(t, b):
            pltpu.make_async_copy(
                table_hbm.at[idx_slice(t)], bufs[b], g_sem[b]).start()

        def gather_wait(t, b):
            pltpu.make_async_copy(
                table_hbm.at[idx_slice(t)], bufs[b], g_sem[b]).wait()

        def store_start(t, b):
            pltpu.make_async_copy(bufs[b], out_slice(t), s_sem[b]).start()

        def store_wait(t, b):
            pltpu.make_async_copy(bufs[b], out_slice(t), s_sem[b]).wait()

        for t in range(LOOKAHEAD):
            gather_start(t, t)
        for t in range(LOOKAHEAD):
            gather_wait(t, t)
            store_start(t, t)
            gather_start(t + LOOKAHEAD, t + LOOKAHEAD)

        @pl.loop(0, n_outer)
        def _(k):
            for j in range(NBUF):
                t = LOOKAHEAD + k * NBUF + j
                b = (LOOKAHEAD + j) % NBUF
                b2 = (b + LOOKAHEAD) % NBUF
                gather_wait(t, b)
                store_start(t, b)
                store_wait(t - LOOKAHEAD, b2)
                gather_start(t + LOOKAHEAD, b2)

        for i in range(LOOKAHEAD):
            t = n_chunks - LOOKAHEAD + i
            b = t % NBUF
            gather_wait(t, b)
            store_start(t, b)

        for i in range(NBUF):
            t = n_chunks - NBUF + i
            store_wait(t, t % NBUF)

    out = gather_kernel(table, flat_idx)
    return out.reshape(batch, seq, hidden)


# speedup vs baseline: 1.0179x; 1.0005x over previous
import functools

import jax
import jax.numpy as jnp
from jax import lax
from jax.experimental import pallas as pl
from jax.experimental.pallas import tpu as pltpu
from jax.experimental.pallas import tpu_sc as plsc

NUM_CORES = 2
NUM_SUBCORES = 16
CHUNK = 128
LOOKAHEAD = 5
NBUF = 2 * LOOKAHEAD


def kernel(input_ids, table):
    batch, seq = input_ids.shape
    vocab, hidden = table.shape
    num_idx = batch * seq
    flat_idx = input_ids.reshape(num_idx)

    num_workers = NUM_CORES * NUM_SUBCORES
    per_worker = num_idx // num_workers
    n_chunks = per_worker // CHUNK
    steady = n_chunks - 2 * LOOKAHEAD
    n_outer = steady // NBUF
    assert steady == n_outer * NBUF

    mesh = plsc.VectorSubcoreMesh(core_axis_name="c", subcore_axis_name="s")

    @functools.partial(
        pl.kernel,
        mesh=mesh,
        out_type=jax.ShapeDtypeStruct((num_idx, hidden), table.dtype),
        scratch_types=[
            pltpu.VMEM((per_worker,), jnp.int32),
        ]
        + [pltpu.VMEM((CHUNK, hidden), jnp.float32)] * NBUF
        + [pltpu.SemaphoreType.DMA] * (2 * NBUF)
        + [pltpu.SemaphoreType.DMA],
        compiler_params=pltpu.CompilerParams(use_tc_tiling_on_sc=False),
    )
    def gather_kernel(table_hbm, idx_hbm, out_hbm, idx_all, *rest):
        bufs = rest[:NBUF]
        g_sem = rest[NBUF:2 * NBUF]
        s_sem = rest[2 * NBUF:3 * NBUF]
        idx_sem = rest[3 * NBUF]

        wid = lax.axis_index("s") * NUM_CORES + lax.axis_index("c")
        base_w = wid * per_worker

        pltpu.async_copy(
            idx_hbm.at[pl.ds(base_w, per_worker)], idx_all, idx_sem).wait()

        def idx_slice(t):
            return idx_all.at[pl.ds(t * CHUNK, CHUNK)]

        def out_slice(t):
            return out_hbm.at[pl.ds(base_w + t * CHUNK, CHUNK)]

        def gather_start(t, b):
            pltpu.make_async_copy(
                table_hbm.at[idx_slice(t)], bufs[b], g_sem[b]).start()

        def gather_wait(t, b):
            pltpu.make_async_copy(
                table_hbm.at[idx_slice(t)], bufs[b], g_sem[b]).wait()

        def store_start(t, b):
            pltpu.make_async_copy(bufs[b], out_slice(t), s_sem[b]).start()

        def store_wait(t, b):
            pltpu.make_async_copy(bufs[b], out_slice(t), s_sem[b]).wait()

        for t in range(LOOKAHEAD):
            gather_start(t, t)
        for t in range(LOOKAHEAD):
            gather_wait(t, t)
            store_start(t, t)
            gather_start(t + LOOKAHEAD, t + LOOKAHEAD)

        @pl.loop(0, n_outer)
        def _(k):
            for j in range(NBUF):
                t = LOOKAHEAD + k * NBUF + j
                b = (LOOKAHEAD + j) % NBUF
                b2 = (b + LOOKAHEAD) % NBUF
                gather_wait(t, b)
                store_start(t, b)
                store_wait(t - LOOKAHEAD, b2)
                gather_start(t + LOOKAHEAD, b2)

        for i in range(LOOKAHEAD):
            t = n_chunks - LOOKAHEAD + i
            b = t % NBUF
            gather_wait(t, b)
            store_start(t, b)

        for i in range(NBUF):
            t = n_chunks - NBUF + i
            store_wait(t, t % NBUF)

    out = gather_kernel(table, flat_idx)
    return out.reshape(batch, seq, hidden)
